# baseline (speedup 1.0000x reference)
import numpy as np

import jax
import jax.numpy as jnp
from jax.experimental import pallas as pl

B = 32
T = 64
D = 128
NF = 32
NE = 14
ROWS_PER_STEP = 2048
GRID = (B * T * 16) // ROWS_PER_STEP

_HI = jax.lax.Precision.HIGHEST


def _gelu(h):
    return 0.5 * h * (1.0 + jax.lax.erf(h * np.float32(1.0 / np.sqrt(2.0))))


def _ln(h, g, b):
    mu = jnp.mean(h, axis=-1, keepdims=True)
    var = jnp.mean((h - mu) * (h - mu), axis=-1, keepdims=True)
    return (h - mu) * jax.lax.rsqrt(var + 1e-5) * g + b


def _main_kernel(x_ref, w0_ref, w1_ref, w2_ref, fw_ref,
                 cb0_ref, lg0_ref, lb0_ref,
                 cb1_ref, lg1_ref, lb1_ref,
                 cb2_ref, lg2_ref, lb2_ref,
                 fb_ref, bdre_ref, bdim_ref, amp_ref):
    i = pl.program_id(0)
    h = x_ref[...]
    h = jnp.dot(h, w0_ref[...], precision=_HI) + cb0_ref[...]
    h = _gelu(_ln(h, lg0_ref[...], lb0_ref[...]))
    h = h.reshape(ROWS_PER_STEP // 4, 1024)
    h = jnp.dot(h, w1_ref[...], precision=_HI) + cb1_ref[...]
    h = _gelu(_ln(h, lg1_ref[...], lb1_ref[...]))
    h = h.reshape(ROWS_PER_STEP // 16, 2048)
    h = jnp.dot(h, w2_ref[...], precision=_HI) + cb2_ref[...]
    h = _gelu(_ln(h, lg2_ref[...], lb2_ref[...]))
    y = jnp.dot(h, fw_ref[...], precision=_HI) + fb_ref[...]
    re = jnp.dot(bdre_ref[...], y, precision=_HI)
    im = jnp.dot(bdim_ref[...], y, precision=_HI)
    a = jnp.sqrt(re * re + im * im)
    ones = jnp.ones((1, a.shape[1]), jnp.float32)
    row = jax.lax.dot_general(ones, a, (((1,), (1,)), ((), ())),
                              precision=_HI) * (1.0 / a.shape[1])
    amp_ref[pl.ds(i, 1), :] = row


def _gate_kernel(amp_ref, wg_ref, gates_ref, load_ref):
    logits = jnp.dot(amp_ref[...], wg_ref[...], precision=_HI)
    lane = jax.lax.broadcasted_iota(jnp.int32, logits.shape, 1)
    neg = jnp.float32(-jnp.inf)
    logits = jnp.where(lane < NE, logits, neg)
    v1 = jnp.max(logits, axis=1, keepdims=True)
    i1 = jnp.min(jnp.where(logits == v1, lane, NE + 1), axis=1, keepdims=True)
    l2 = jnp.where(lane == i1, neg, logits)
    v2 = jnp.max(l2, axis=1, keepdims=True)
    i2 = jnp.min(jnp.where(l2 == v2, lane, NE + 1), axis=1, keepdims=True)
    e = jnp.exp(v2 - v1)
    g1 = 1.0 / (1.0 + e)
    g2 = e / (1.0 + e)
    gates = (jnp.where(lane == i1, g1, 0.0)
             + jnp.where(lane == i2, g2, 0.0))
    gates_ref[...] = gates
    load_ref[...] = jnp.sum((gates > 0.0).astype(jnp.int32), axis=0,
                            keepdims=True)


def kernel(x, training, conv_w0, conv_b0, ln_g0, ln_b0,
           conv_w1, conv_b1, ln_g1, ln_b1,
           conv_w2, conv_b2, ln_g2, ln_b2,
           fuse_w, fuse_b, w_gate):
    bt = B * T
    xm = x.reshape(bt, 2, 2, 2, 2, 2, 2, D)
    xm = xm.transpose(0, 1, 4, 2, 5, 3, 6, 7).reshape(bt * 16, 4 * D)

    w0p = conv_w0.transpose(2, 3, 1, 0).reshape(4 * D, 2 * D)
    w1p = conv_w1.transpose(2, 3, 1, 0).reshape(8 * D, 4 * D)
    w2p = conv_w2.transpose(2, 3, 1, 0).reshape(16 * D, 8 * D)

    r2 = lambda v: v.reshape(1, -1)

    tt = np.arange(T)
    kk = np.arange(1, NF + 1)
    ang = 2.0 * np.pi * np.outer(kk, tt) / T
    fre = (np.cos(ang) / np.sqrt(T)).astype(np.float32)
    fim = (-np.sin(ang) / np.sqrt(T)).astype(np.float32)
    bdre = np.zeros((2 * NF, 2 * T), np.float32)
    bdim = np.zeros((2 * NF, 2 * T), np.float32)
    for r in range(2):
        bdre[r * NF:(r + 1) * NF, r * T:(r + 1) * T] = fre
        bdim[r * NF:(r + 1) * NF, r * T:(r + 1) * T] = fim
    bdre = jnp.asarray(bdre)
    bdim = jnp.asarray(bdim)

    row_spec = pl.BlockSpec((ROWS_PER_STEP, 4 * D), lambda i: (i, 0))
    full = lambda a: pl.BlockSpec(a.shape, lambda i: (0,) * a.ndim)

    ins = (xm, w0p, w1p, w2p, fuse_w,
           r2(conv_b0), r2(ln_g0), r2(ln_b0),
           r2(conv_b1), r2(ln_g1), r2(ln_b1),
           r2(conv_b2), r2(ln_g2), r2(ln_b2),
           r2(fuse_b), bdre, bdim)
    amp16 = pl.pallas_call(
        _main_kernel,
        grid=(GRID,),
        in_specs=[row_spec] + [full(a) for a in ins[1:]],
        out_specs=pl.BlockSpec((GRID, 2 * NF), lambda i: (0, 0)),
        out_shape=jax.ShapeDtypeStruct((GRID, 2 * NF), jnp.float32),
    )(*ins)

    amp = amp16.reshape(B, NF)
    wg_pad = jnp.zeros((NF, 128), jnp.float32).at[:, :NE].set(w_gate)
    gates_pad, load_pad = pl.pallas_call(
        _gate_kernel,
        out_shape=(jax.ShapeDtypeStruct((B, 128), jnp.float32),
                   jax.ShapeDtypeStruct((1, 128), jnp.int32)),
    )(amp, wg_pad)
    return gates_pad[:, :NE], load_pad[0, :NE]


# speedup vs baseline: 1.3642x; 1.3642x over previous
import numpy as np

import jax
import jax.numpy as jnp
from jax.experimental import pallas as pl

B = 32
T = 64
D = 128
NF = 32
NE = 14
ROWS_PER_STEP = 2048
GRID = (B * T * 16) // ROWS_PER_STEP

_HI = jax.lax.Precision.HIGHEST
_F32 = jnp.float32
_BF16 = jnp.bfloat16


def _split(a):
    hi = a.astype(_BF16)
    lo = (a - hi.astype(_F32)).astype(_BF16)
    return hi, lo


def _dot3(a, b_hi, b_lo):
    a_hi, a_lo = _split(a)
    acc = jnp.dot(a_hi, b_hi, preferred_element_type=_F32)
    acc += jnp.dot(a_hi, b_lo, preferred_element_type=_F32)
    acc += jnp.dot(a_lo, b_hi, preferred_element_type=_F32)
    return acc


def _gelu(h):
    return 0.5 * h * (1.0 + jax.lax.erf(h * np.float32(1.0 / np.sqrt(2.0))))


def _ln(h, g, b):
    mu = jnp.mean(h, axis=-1, keepdims=True)
    var = jnp.mean((h - mu) * (h - mu), axis=-1, keepdims=True)
    return (h - mu) * jax.lax.rsqrt(var + 1e-5) * g + b


def _main_kernel(x_ref,
                 w0h_ref, w0l_ref, w1h_ref, w1l_ref,
                 w2h_ref, w2l_ref, fwh_ref, fwl_ref,
                 cb0_ref, lg0_ref, lb0_ref,
                 cb1_ref, lg1_ref, lb1_ref,
                 cb2_ref, lg2_ref, lb2_ref,
                 fb_ref, bdre_ref, bdim_ref, amp_ref):
    i = pl.program_id(0)
    h = x_ref[...]
    h = _dot3(h, w0h_ref[...], w0l_ref[...]) + cb0_ref[...]
    h = _gelu(_ln(h, lg0_ref[...], lb0_ref[...]))
    h = h.reshape(ROWS_PER_STEP // 4, 1024)
    h = _dot3(h, w1h_ref[...], w1l_ref[...]) + cb1_ref[...]
    h = _gelu(_ln(h, lg1_ref[...], lb1_ref[...]))
    h = h.reshape(ROWS_PER_STEP // 16, 2048)
    h = _dot3(h, w2h_ref[...], w2l_ref[...]) + cb2_ref[...]
    h = _gelu(_ln(h, lg2_ref[...], lb2_ref[...]))
    y = _dot3(h, fwh_ref[...], fwl_ref[...]) + fb_ref[...]
    re = jnp.dot(bdre_ref[...], y, precision=_HI)
    im = jnp.dot(bdim_ref[...], y, precision=_HI)
    a = jnp.sqrt(re * re + im * im)
    ones = jnp.ones((1, a.shape[1]), _F32)
    row = jax.lax.dot_general(ones, a, (((1,), (1,)), ((), ())),
                              precision=_HI) * (1.0 / a.shape[1])
    amp_ref[pl.ds(i, 1), :] = row


def _gate_kernel(amp_ref, wg_ref, gates_ref, load_ref):
    logits = jnp.dot(amp_ref[...], wg_ref[...], precision=_HI)
    lane = jax.lax.broadcasted_iota(jnp.int32, logits.shape, 1)
    neg = jnp.float32(-jnp.inf)
    logits = jnp.where(lane < NE, logits, neg)
    v1 = jnp.max(logits, axis=1, keepdims=True)
    i1 = jnp.min(jnp.where(logits == v1, lane, NE + 1), axis=1, keepdims=True)
    l2 = jnp.where(lane == i1, neg, logits)
    v2 = jnp.max(l2, axis=1, keepdims=True)
    i2 = jnp.min(jnp.where(l2 == v2, lane, NE + 1), axis=1, keepdims=True)
    e = jnp.exp(v2 - v1)
    g1 = 1.0 / (1.0 + e)
    g2 = e / (1.0 + e)
    gates = (jnp.where(lane == i1, g1, 0.0)
             + jnp.where(lane == i2, g2, 0.0))
    gates_ref[...] = gates
    load_ref[...] = jnp.sum((gates > 0.0).astype(jnp.int32), axis=0,
                            keepdims=True)


def kernel(x, training, conv_w0, conv_b0, ln_g0, ln_b0,
           conv_w1, conv_b1, ln_g1, ln_b1,
           conv_w2, conv_b2, ln_g2, ln_b2,
           fuse_w, fuse_b, w_gate):
    bt = B * T
    xm = x.reshape(bt, 2, 2, 2, 2, 2, 2, D)
    xm = xm.transpose(0, 1, 4, 2, 5, 3, 6, 7).reshape(bt * 16, 4 * D)

    w0p = conv_w0.transpose(2, 3, 1, 0).reshape(4 * D, 2 * D)
    w1p = conv_w1.transpose(2, 3, 1, 0).reshape(8 * D, 4 * D)
    w2p = conv_w2.transpose(2, 3, 1, 0).reshape(16 * D, 8 * D)
    w0h, w0l = _split(w0p)
    w1h, w1l = _split(w1p)
    w2h, w2l = _split(w2p)
    fwh, fwl = _split(fuse_w)

    r2 = lambda v: v.reshape(1, -1)

    tt = np.arange(T)
    kk = np.arange(1, NF + 1)
    ang = 2.0 * np.pi * np.outer(kk, tt) / T
    fre = (np.cos(ang) / np.sqrt(T)).astype(np.float32)
    fim = (-np.sin(ang) / np.sqrt(T)).astype(np.float32)
    bdre = np.zeros((2 * NF, 2 * T), np.float32)
    bdim = np.zeros((2 * NF, 2 * T), np.float32)
    for r in range(2):
        bdre[r * NF:(r + 1) * NF, r * T:(r + 1) * T] = fre
        bdim[r * NF:(r + 1) * NF, r * T:(r + 1) * T] = fim
    bdre = jnp.asarray(bdre)
    bdim = jnp.asarray(bdim)

    row_spec = pl.BlockSpec((ROWS_PER_STEP, 4 * D), lambda i: (i, 0))
    full = lambda a: pl.BlockSpec(a.shape, lambda i: (0,) * a.ndim)

    ins = (xm, w0h, w0l, w1h, w1l, w2h, w2l, fwh, fwl,
           r2(conv_b0), r2(ln_g0), r2(ln_b0),
           r2(conv_b1), r2(ln_g1), r2(ln_b1),
           r2(conv_b2), r2(ln_g2), r2(ln_b2),
           r2(fuse_b), bdre, bdim)
    amp16 = pl.pallas_call(
        _main_kernel,
        grid=(GRID,),
        in_specs=[row_spec] + [full(a) for a in ins[1:]],
        out_specs=pl.BlockSpec((GRID, 2 * NF), lambda i: (0, 0)),
        out_shape=jax.ShapeDtypeStruct((GRID, 2 * NF), jnp.float32),
    )(*ins)

    amp = amp16.reshape(B, NF)
    wg_pad = jnp.zeros((NF, 128), jnp.float32).at[:, :NE].set(w_gate)
    gates_pad, load_pad = pl.pallas_call(
        _gate_kernel,
        out_shape=(jax.ShapeDtypeStruct((B, 128), jnp.float32),
                   jax.ShapeDtypeStruct((1, 128), jnp.int32)),
    )(amp, wg_pad)
    return gates_pad[:, :NE], load_pad[0, :NE]


# speedup vs baseline: 2.0276x; 1.4863x over previous
import numpy as np

import jax
import jax.numpy as jnp
from jax.experimental import pallas as pl

B = 32
T = 64
D = 128
NF = 32
NE = 14
NB = 128
GRID = (B * T) // NB

_HI = jax.lax.Precision.HIGHEST
_F32 = jnp.float32
_BF16 = jnp.bfloat16


def _split(a):
    hi = a.astype(_BF16)
    lo = (a - hi.astype(_F32)).astype(_BF16)
    return hi, lo


def _dot3(a, b_hi, b_lo):
    a_hi, a_lo = _split(a)
    acc = jnp.dot(a_hi, b_hi, preferred_element_type=_F32)
    acc += jnp.dot(a_hi, b_lo, preferred_element_type=_F32)
    acc += jnp.dot(a_lo, b_hi, preferred_element_type=_F32)
    return acc


def _gelu(h):
    return 0.5 * h * (1.0 + jax.lax.erf(h * np.float32(1.0 / np.sqrt(2.0))))


def _ln(h, g, b):
    mu = jnp.mean(h, axis=-1, keepdims=True)
    var = jnp.mean((h - mu) * (h - mu), axis=-1, keepdims=True)
    return (h - mu) * jax.lax.rsqrt(var + 1e-5) * g + b


def _main_kernel(x_ref,
                 w0h_ref, w0l_ref, w1h_ref, w1l_ref,
                 w2h_ref, w2l_ref, fwh_ref, fwl_ref,
                 cb0_ref, lg0_ref, lb0_ref,
                 cb1_ref, lg1_ref, lb1_ref,
                 cb2_ref, lg2_ref, lb2_ref,
                 fb_ref, bdre_ref, bdim_ref, amp_ref):
    i = pl.program_id(0)
    x = x_ref[...]
    xv = x.reshape(NB, 2, 2, 2, 2, 2, 2, D)
    cb0, lg0, lb0 = cb0_ref[...], lg0_ref[...], lb0_ref[...]
    w0h, w0l = w0h_ref[...], w0l_ref[...]
    h0 = {}
    for a0 in (0, 1):
        for b0 in (0, 1):
            parts = [xv[:, :, a0, i0, :, b0, j0, :].reshape(NB * 4, D)
                     for i0 in (0, 1) for j0 in (0, 1)]
            xc = jnp.concatenate(parts, axis=1)
            h0[(a0, b0)] = _gelu(_ln(_dot3(xc, w0h, w0l) + cb0, lg0, lb0))
    h1 = _dot3(h0[(0, 0)], w1h_ref[0], w1l_ref[0])
    h1 += _dot3(h0[(0, 1)], w1h_ref[1], w1l_ref[1])
    h1 += _dot3(h0[(1, 0)], w1h_ref[2], w1l_ref[2])
    h1 += _dot3(h0[(1, 1)], w1h_ref[3], w1l_ref[3])
    h1 = _gelu(_ln(h1 + cb1_ref[...], lg1_ref[...], lb1_ref[...]))
    h1v = h1.reshape(NB, 2, 2, 512)
    h2 = _dot3(h1v[:, 0, 0, :], w2h_ref[0], w2l_ref[0])
    h2 += _dot3(h1v[:, 0, 1, :], w2h_ref[1], w2l_ref[1])
    h2 += _dot3(h1v[:, 1, 0, :], w2h_ref[2], w2l_ref[2])
    h2 += _dot3(h1v[:, 1, 1, :], w2h_ref[3], w2l_ref[3])
    h2 = _gelu(_ln(h2 + cb2_ref[...], lg2_ref[...], lb2_ref[...]))
    y = _dot3(h2, fwh_ref[...], fwl_ref[...]) + fb_ref[...]
    re = jnp.dot(bdre_ref[...], y, precision=_HI)
    im = jnp.dot(bdim_ref[...], y, precision=_HI)
    a = jnp.sqrt(re * re + im * im)
    ones = jnp.ones((1, a.shape[1]), _F32)
    row = jax.lax.dot_general(ones, a, (((1,), (1,)), ((), ())),
                              precision=_HI) * (1.0 / a.shape[1])
    amp_ref[pl.ds(i, 1), :] = row


def _gate_kernel(amp_ref, wg_ref, gates_ref, load_ref):
    logits = jnp.dot(amp_ref[...], wg_ref[...], precision=_HI)
    lane = jax.lax.broadcasted_iota(jnp.int32, logits.shape, 1)
    neg = jnp.float32(-jnp.inf)
    v1 = jnp.max(logits, axis=1, keepdims=True)
    i1 = jnp.min(jnp.where(logits == v1, lane, NE + 1), axis=1, keepdims=True)
    l2 = jnp.where(lane == i1, neg, logits)
    v2 = jnp.max(l2, axis=1, keepdims=True)
    i2 = jnp.min(jnp.where(l2 == v2, lane, NE + 1), axis=1, keepdims=True)
    e = jnp.exp(v2 - v1)
    g1 = 1.0 / (1.0 + e)
    g2 = e / (1.0 + e)
    gates = (jnp.where(lane == i1, g1, 0.0)
             + jnp.where(lane == i2, g2, 0.0))
    gates_ref[...] = gates
    load_ref[...] = jnp.sum((gates > 0.0).astype(jnp.int32), axis=0,
                            keepdims=True)


def kernel(x, training, conv_w0, conv_b0, ln_g0, ln_b0,
           conv_w1, conv_b1, ln_g1, ln_b1,
           conv_w2, conv_b2, ln_g2, ln_b2,
           fuse_w, fuse_b, w_gate):
    bt = B * T
    xr = x.reshape(bt * 64, D)

    w0p = conv_w0.transpose(2, 3, 1, 0).reshape(4 * D, 2 * D)
    w1q = conv_w1.transpose(2, 3, 1, 0).reshape(4, 2 * D, 4 * D)
    w2q = conv_w2.transpose(2, 3, 1, 0).reshape(4, 4 * D, 8 * D)
    w0h, w0l = _split(w0p)
    w1h, w1l = _split(w1q)
    w2h, w2l = _split(w2q)
    fwh, fwl = _split(fuse_w)

    r2 = lambda v: v.reshape(1, -1)

    tt = np.arange(T)
    kk = np.arange(1, NF + 1)
    ang = 2.0 * np.pi * np.outer(kk, tt) / T
    fre = (np.cos(ang) / np.sqrt(T)).astype(np.float32)
    fim = (-np.sin(ang) / np.sqrt(T)).astype(np.float32)
    bdre = np.zeros((2 * NF, 2 * T), np.float32)
    bdim = np.zeros((2 * NF, 2 * T), np.float32)
    for r in range(2):
        bdre[r * NF:(r + 1) * NF, r * T:(r + 1) * T] = fre
        bdim[r * NF:(r + 1) * NF, r * T:(r + 1) * T] = fim
    bdre = jnp.asarray(bdre)
    bdim = jnp.asarray(bdim)

    row_spec = pl.BlockSpec((NB * 64, D), lambda i: (i, 0))
    full = lambda a: pl.BlockSpec(a.shape, lambda i: (0,) * a.ndim)

    ins = (xr, w0h, w0l, w1h, w1l, w2h, w2l, fwh, fwl,
           r2(conv_b0), r2(ln_g0), r2(ln_b0),
           r2(conv_b1), r2(ln_g1), r2(ln_b1),
           r2(conv_b2), r2(ln_g2), r2(ln_b2),
           r2(fuse_b), bdre, bdim)
    amp16 = pl.pallas_call(
        _main_kernel,
        grid=(GRID,),
        in_specs=[row_spec] + [full(a) for a in ins[1:]],
        out_specs=pl.BlockSpec((GRID, 2 * NF), lambda i: (0, 0)),
        out_shape=jax.ShapeDtypeStruct((GRID, 2 * NF), jnp.float32),
    )(*ins)

    amp = amp16.reshape(B, NF)
    gates, load = pl.pallas_call(
        _gate_kernel,
        out_shape=(jax.ShapeDtypeStruct((B, NE), jnp.float32),
                   jax.ShapeDtypeStruct((1, NE), jnp.int32)),
    )(amp, w_gate)
    return gates, load.reshape(NE)


# speedup vs baseline: 2.1417x; 1.0563x over previous
import numpy as np

import jax
import jax.numpy as jnp
from jax.experimental import pallas as pl

B = 32
T = 64
D = 128
NF = 32
NE = 14
NB = 256
GRID = (B * T) // NB

_HI = jax.lax.Precision.HIGHEST
_F32 = jnp.float32
_BF16 = jnp.bfloat16


def _split(a):
    hi = a.astype(_BF16)
    lo = (a - hi.astype(_F32)).astype(_BF16)
    return hi, lo


def _dot3(a, b_hi, b_lo):
    a_hi, a_lo = _split(a)
    acc = jnp.dot(a_hi, b_hi, preferred_element_type=_F32)
    acc += jnp.dot(a_hi, b_lo, preferred_element_type=_F32)
    acc += jnp.dot(a_lo, b_hi, preferred_element_type=_F32)
    return acc


def _gelu(h):
    return 0.5 * h * (1.0 + jax.lax.erf(h * np.float32(1.0 / np.sqrt(2.0))))


def _ln(h, g, b):
    mu = jnp.mean(h, axis=-1, keepdims=True)
    var = jnp.mean((h - mu) * (h - mu), axis=-1, keepdims=True)
    return (h - mu) * jax.lax.rsqrt(var + 1e-5) * g + b


def _main_kernel(x_ref,
                 w0h_ref, w0l_ref, w1h_ref, w1l_ref,
                 w2h_ref, w2l_ref, fwh_ref, fwl_ref,
                 cb0_ref, lg0_ref, lb0_ref,
                 cb1_ref, lg1_ref, lb1_ref,
                 cb2_ref, lg2_ref, lb2_ref,
                 fb_ref, bdre_ref, bdim_ref, amp_ref):
    i = pl.program_id(0)
    x = x_ref[...]
    xv = x.reshape(NB, 2, 2, 2, 2, 2, 2, D)
    cb0, lg0, lb0 = cb0_ref[...], lg0_ref[...], lb0_ref[...]
    w0h, w0l = w0h_ref[...], w0l_ref[...]
    h0 = {}
    for a0 in (0, 1):
        for b0 in (0, 1):
            parts = [xv[:, :, a0, i0, :, b0, j0, :].reshape(NB * 4, D)
                     for i0 in (0, 1) for j0 in (0, 1)]
            xc = jnp.concatenate(parts, axis=1)
            h0[(a0, b0)] = _gelu(_ln(_dot3(xc, w0h, w0l) + cb0, lg0, lb0))
    h1 = _dot3(h0[(0, 0)], w1h_ref[0], w1l_ref[0])
    h1 += _dot3(h0[(0, 1)], w1h_ref[1], w1l_ref[1])
    h1 += _dot3(h0[(1, 0)], w1h_ref[2], w1l_ref[2])
    h1 += _dot3(h0[(1, 1)], w1h_ref[3], w1l_ref[3])
    h1 = _gelu(_ln(h1 + cb1_ref[...], lg1_ref[...], lb1_ref[...]))
    h1v = h1.reshape(NB, 2, 2, 512)
    h2 = _dot3(h1v[:, 0, 0, :], w2h_ref[0], w2l_ref[0])
    h2 += _dot3(h1v[:, 0, 1, :], w2h_ref[1], w2l_ref[1])
    h2 += _dot3(h1v[:, 1, 0, :], w2h_ref[2], w2l_ref[2])
    h2 += _dot3(h1v[:, 1, 1, :], w2h_ref[3], w2l_ref[3])
    h2 = _gelu(_ln(h2 + cb2_ref[...], lg2_ref[...], lb2_ref[...]))
    y = _dot3(h2, fwh_ref[...], fwl_ref[...]) + fb_ref[...]
    re = jnp.dot(bdre_ref[...], y, precision=_HI)
    im = jnp.dot(bdim_ref[...], y, precision=_HI)
    a = jnp.sqrt(re * re + im * im)
    ones = jnp.ones((1, a.shape[1]), _F32)
    row = jax.lax.dot_general(ones, a, (((1,), (1,)), ((), ())),
                              precision=_HI) * (1.0 / a.shape[1])
    amp_ref[pl.ds(i, 1), :] = row


def _gate_kernel(amp_ref, wg_ref, gates_ref, load_ref):
    logits = jnp.dot(amp_ref[...], wg_ref[...], precision=_HI)
    lane = jax.lax.broadcasted_iota(jnp.int32, logits.shape, 1)
    neg = jnp.float32(-jnp.inf)
    v1 = jnp.max(logits, axis=1, keepdims=True)
    i1 = jnp.min(jnp.where(logits == v1, lane, NE + 1), axis=1, keepdims=True)
    l2 = jnp.where(lane == i1, neg, logits)
    v2 = jnp.max(l2, axis=1, keepdims=True)
    i2 = jnp.min(jnp.where(l2 == v2, lane, NE + 1), axis=1, keepdims=True)
    e = jnp.exp(v2 - v1)
    g1 = 1.0 / (1.0 + e)
    g2 = e / (1.0 + e)
    gates = (jnp.where(lane == i1, g1, 0.0)
             + jnp.where(lane == i2, g2, 0.0))
    gates_ref[...] = gates
    load_ref[...] = jnp.sum((gates > 0.0).astype(jnp.int32), axis=0,
                            keepdims=True)


def kernel(x, training, conv_w0, conv_b0, ln_g0, ln_b0,
           conv_w1, conv_b1, ln_g1, ln_b1,
           conv_w2, conv_b2, ln_g2, ln_b2,
           fuse_w, fuse_b, w_gate):
    bt = B * T
    xr = x.reshape(bt * 64, D)

    w0p = conv_w0.transpose(2, 3, 1, 0).reshape(4 * D, 2 * D)
    w1q = conv_w1.transpose(2, 3, 1, 0).reshape(4, 2 * D, 4 * D)
    w2q = conv_w2.transpose(2, 3, 1, 0).reshape(4, 4 * D, 8 * D)
    w0h, w0l = _split(w0p)
    w1h, w1l = _split(w1q)
    w2h, w2l = _split(w2q)
    fwh, fwl = _split(fuse_w)

    r2 = lambda v: v.reshape(1, -1)

    nbb = NB // T
    tt = np.arange(T)
    kk = np.arange(1, NF + 1)
    ang = 2.0 * np.pi * np.outer(kk, tt) / T
    fre = (np.cos(ang) / np.sqrt(T)).astype(np.float32)
    fim = (-np.sin(ang) / np.sqrt(T)).astype(np.float32)
    bdre = np.zeros((nbb * NF, nbb * T), np.float32)
    bdim = np.zeros((nbb * NF, nbb * T), np.float32)
    for r in range(nbb):
        bdre[r * NF:(r + 1) * NF, r * T:(r + 1) * T] = fre
        bdim[r * NF:(r + 1) * NF, r * T:(r + 1) * T] = fim
    bdre = jnp.asarray(bdre)
    bdim = jnp.asarray(bdim)

    row_spec = pl.BlockSpec((NB * 64, D), lambda i: (i, 0))
    full = lambda a: pl.BlockSpec(a.shape, lambda i: (0,) * a.ndim)

    ins = (xr, w0h, w0l, w1h, w1l, w2h, w2l, fwh, fwl,
           r2(conv_b0), r2(ln_g0), r2(ln_b0),
           r2(conv_b1), r2(ln_g1), r2(ln_b1),
           r2(conv_b2), r2(ln_g2), r2(ln_b2),
           r2(fuse_b), bdre, bdim)
    amp16 = pl.pallas_call(
        _main_kernel,
        grid=(GRID,),
        in_specs=[row_spec] + [full(a) for a in ins[1:]],
        out_specs=pl.BlockSpec((GRID, (NB // T) * NF), lambda i: (0, 0)),
        out_shape=jax.ShapeDtypeStruct((GRID, (NB // T) * NF), jnp.float32),
    )(*ins)

    amp = amp16.reshape(B, NF)
    gates, load = pl.pallas_call(
        _gate_kernel,
        out_shape=(jax.ShapeDtypeStruct((B, NE), jnp.float32),
                   jax.ShapeDtypeStruct((1, NE), jnp.int32)),
    )(amp, w_gate)
    return gates, load.reshape(NE)


# speedup vs baseline: 2.1499x; 1.0038x over previous
import numpy as np

import jax
import jax.numpy as jnp
from jax.experimental import pallas as pl

B = 32
T = 64
D = 128
NF = 32
NE = 14
NB = 256
GRID = (B * T) // NB

_HI = jax.lax.Precision.HIGHEST
_F32 = jnp.float32
_BF16 = jnp.bfloat16


def _split(a):
    hi = a.astype(_BF16)
    lo = (a - hi.astype(_F32)).astype(_BF16)
    return hi, lo


def _dot3(a, b_hi, b_lo):
    a_hi, a_lo = _split(a)
    acc = jnp.dot(a_hi, b_hi, preferred_element_type=_F32)
    acc += jnp.dot(a_hi, b_lo, preferred_element_type=_F32)
    acc += jnp.dot(a_lo, b_hi, preferred_element_type=_F32)
    return acc


def _gelu(h):
    return 0.5 * h * (1.0 + jax.lax.erf(h * np.float32(1.0 / np.sqrt(2.0))))


def _ln(h, g, b):
    c = h.shape[1]
    ones = jnp.ones((1, c), _F32)
    nt = (((1,), (1,)), ((), ()))
    s1 = jax.lax.dot_general(h, ones, nt, precision=_HI)
    s2 = jax.lax.dot_general(h * h, ones, nt, precision=_HI)
    mu = s1 * (1.0 / c)
    var = s2 * (1.0 / c) - mu * mu
    return (h - mu) * jax.lax.rsqrt(var + 1e-5) * g + b


def _main_kernel(x_ref,
                 w0h_ref, w0l_ref, w1h_ref, w1l_ref,
                 w2h_ref, w2l_ref, fwh_ref, fwl_ref,
                 cb0_ref, lg0_ref, lb0_ref,
                 cb1_ref, lg1_ref, lb1_ref,
                 cb2_ref, lg2_ref, lb2_ref,
                 fb_ref, bdre_ref, bdim_ref, amp_ref):
    i = pl.program_id(0)
    x = x_ref[...]
    xv = x.reshape(NB, 2, 2, 2, 2, 2, 2, D)
    cb0, lg0, lb0 = cb0_ref[...], lg0_ref[...], lb0_ref[...]
    w0h, w0l = w0h_ref[...], w0l_ref[...]
    h0 = {}
    for a0 in (0, 1):
        for b0 in (0, 1):
            parts = [xv[:, :, a0, i0, :, b0, j0, :].reshape(NB * 4, D)
                     for i0 in (0, 1) for j0 in (0, 1)]
            xc = jnp.concatenate(parts, axis=1)
            h0[(a0, b0)] = _gelu(_ln(_dot3(xc, w0h, w0l) + cb0, lg0, lb0))
    h1 = _dot3(h0[(0, 0)], w1h_ref[0], w1l_ref[0])
    h1 += _dot3(h0[(0, 1)], w1h_ref[1], w1l_ref[1])
    h1 += _dot3(h0[(1, 0)], w1h_ref[2], w1l_ref[2])
    h1 += _dot3(h0[(1, 1)], w1h_ref[3], w1l_ref[3])
    h1 = _gelu(_ln(h1 + cb1_ref[...], lg1_ref[...], lb1_ref[...]))
    h1v = h1.reshape(NB, 2, 2, 512)
    h2 = _dot3(h1v[:, 0, 0, :], w2h_ref[0], w2l_ref[0])
    h2 += _dot3(h1v[:, 0, 1, :], w2h_ref[1], w2l_ref[1])
    h2 += _dot3(h1v[:, 1, 0, :], w2h_ref[2], w2l_ref[2])
    h2 += _dot3(h1v[:, 1, 1, :], w2h_ref[3], w2l_ref[3])
    h2 = _gelu(_ln(h2 + cb2_ref[...], lg2_ref[...], lb2_ref[...]))
    y = _dot3(h2, fwh_ref[...], fwl_ref[...]) + fb_ref[...]
    y_hi, y_lo = _split(y)
    bd_hi, bd_lo = bdre_ref[...], bdim_ref[...]
    ri = jnp.dot(bd_hi, y_hi, preferred_element_type=_F32)
    ri += jnp.dot(bd_hi, y_lo, preferred_element_type=_F32)
    ri += jnp.dot(bd_lo, y_hi, preferred_element_type=_F32)
    half = ri.shape[0] // 2
    re = ri[:half]
    im = ri[half:]
    a = jnp.sqrt(re * re + im * im)
    ones = jnp.ones((1, a.shape[1]), _F32)
    row = jax.lax.dot_general(ones, a, (((1,), (1,)), ((), ())),
                              precision=_HI) * (1.0 / a.shape[1])
    amp_ref[pl.ds(i, 1), :] = row


def _gate_kernel(amp_ref, wg_ref, gates_ref, load_ref):
    logits = jnp.dot(amp_ref[...], wg_ref[...], precision=_HI)
    lane = jax.lax.broadcasted_iota(jnp.int32, logits.shape, 1)
    neg = jnp.float32(-jnp.inf)
    v1 = jnp.max(logits, axis=1, keepdims=True)
    i1 = jnp.min(jnp.where(logits == v1, lane, NE + 1), axis=1, keepdims=True)
    l2 = jnp.where(lane == i1, neg, logits)
    v2 = jnp.max(l2, axis=1, keepdims=True)
    i2 = jnp.min(jnp.where(l2 == v2, lane, NE + 1), axis=1, keepdims=True)
    e = jnp.exp(v2 - v1)
    g1 = 1.0 / (1.0 + e)
    g2 = e / (1.0 + e)
    gates = (jnp.where(lane == i1, g1, 0.0)
             + jnp.where(lane == i2, g2, 0.0))
    gates_ref[...] = gates
    load_ref[...] = jnp.sum((gates > 0.0).astype(jnp.int32), axis=0,
                            keepdims=True)


def kernel(x, training, conv_w0, conv_b0, ln_g0, ln_b0,
           conv_w1, conv_b1, ln_g1, ln_b1,
           conv_w2, conv_b2, ln_g2, ln_b2,
           fuse_w, fuse_b, w_gate):
    bt = B * T
    xr = x.reshape(bt * 64, D)

    w0p = conv_w0.transpose(2, 3, 1, 0).reshape(4 * D, 2 * D)
    w1q = conv_w1.transpose(2, 3, 1, 0).reshape(4, 2 * D, 4 * D)
    w2q = conv_w2.transpose(2, 3, 1, 0).reshape(4, 4 * D, 8 * D)
    w0h, w0l = _split(w0p)
    w1h, w1l = _split(w1q)
    w2h, w2l = _split(w2q)
    fwh, fwl = _split(fuse_w)

    r2 = lambda v: v.reshape(1, -1)

    nbb = NB // T
    tt = np.arange(T)
    kk = np.arange(1, NF + 1)
    ang = 2.0 * np.pi * np.outer(kk, tt) / T
    fre = (np.cos(ang) / np.sqrt(T)).astype(np.float32)
    fim = (-np.sin(ang) / np.sqrt(T)).astype(np.float32)
    bdre = np.zeros((nbb * NF, nbb * T), np.float32)
    bdim = np.zeros((nbb * NF, nbb * T), np.float32)
    for r in range(nbb):
        bdre[r * NF:(r + 1) * NF, r * T:(r + 1) * T] = fre
        bdim[r * NF:(r + 1) * NF, r * T:(r + 1) * T] = fim
    bds = jnp.asarray(np.concatenate([bdre, bdim], axis=0))
    bdre, bdim = _split(bds)

    row_spec = pl.BlockSpec((NB * 64, D), lambda i: (i, 0))
    full = lambda a: pl.BlockSpec(a.shape, lambda i: (0,) * a.ndim)

    ins = (xr, w0h, w0l, w1h, w1l, w2h, w2l, fwh, fwl,
           r2(conv_b0), r2(ln_g0), r2(ln_b0),
           r2(conv_b1), r2(ln_g1), r2(ln_b1),
           r2(conv_b2), r2(ln_g2), r2(ln_b2),
           r2(fuse_b), bdre, bdim)
    amp16 = pl.pallas_call(
        _main_kernel,
        grid=(GRID,),
        in_specs=[row_spec] + [full(a) for a in ins[1:]],
        out_specs=pl.BlockSpec((GRID, (NB // T) * NF), lambda i: (0, 0)),
        out_shape=jax.ShapeDtypeStruct((GRID, (NB // T) * NF), jnp.float32),
    )(*ins)

    amp = amp16.reshape(B, NF)
    gates, load = pl.pallas_call(
        _gate_kernel,
        out_shape=(jax.ShapeDtypeStruct((B, NE), jnp.float32),
                   jax.ShapeDtypeStruct((1, NE), jnp.int32)),
    )(amp, w_gate)
    return gates, load.reshape(NE)
